# Initial kernel scaffold; baseline (speedup 1.0000x reference)
import jax
import jax.numpy as jnp
from jax.experimental import pallas as pl


def kernel(x, edge_index, weight, W_ih, W_hh, b_ih, b_hh):
    raise NotImplementedError("write your pallas kernel here")


# speedup vs baseline: 1.4852x; 1.4852x over previous
import functools

import jax
import jax.numpy as jnp
from jax import lax
from jax.experimental import pallas as pl
from jax.experimental.pallas import tpu as pltpu
from jax.experimental.pallas import tpu_sc as plsc

_LANES = 16


def _matmul_body(x_ref, w_ref, o_ref):
    o_ref[...] = jnp.dot(x_ref[...], w_ref[...],
                         preferred_element_type=jnp.float32)


def _tc_matmul(x, w, bn):
    n, k = x.shape
    kk, m = w.shape
    return pl.pallas_call(
        _matmul_body,
        grid=(n // bn,),
        in_specs=[
            pl.BlockSpec((bn, k), lambda i: (i, 0)),
            pl.BlockSpec((kk, m), lambda i: (0, 0)),
        ],
        out_specs=pl.BlockSpec((bn, m), lambda i: (i, 0)),
        out_shape=jax.ShapeDtypeStruct((n, m), jnp.float32),
    )(x, w)


def _gru_body(c, agg_ref, h_ref, wih_ref, whh_ref, bih_ref, bhh_ref, o_ref):
    h = h_ref[...]
    gi = jnp.dot(agg_ref[...], wih_ref[...],
                 preferred_element_type=jnp.float32) + bih_ref[...]
    gh = jnp.dot(h, whh_ref[...],
                 preferred_element_type=jnp.float32) + bhh_ref[...]
    r = jax.nn.sigmoid(gi[:, :c] + gh[:, :c])
    z = jax.nn.sigmoid(gi[:, c:2 * c] + gh[:, c:2 * c])
    nn = jnp.tanh(gi[:, 2 * c:] + r * gh[:, 2 * c:])
    o_ref[...] = (1.0 - z) * nn + z * h


def _tc_gru(agg, h, wih_t, whh_t, bih, bhh, bn):
    n, c = h.shape
    g3 = wih_t.shape[1]
    return pl.pallas_call(
        functools.partial(_gru_body, c),
        grid=(n // bn,),
        in_specs=[
            pl.BlockSpec((bn, c), lambda i: (i, 0)),
            pl.BlockSpec((bn, c), lambda i: (i, 0)),
            pl.BlockSpec((c, g3), lambda i: (0, 0)),
            pl.BlockSpec((c, g3), lambda i: (0, 0)),
            pl.BlockSpec((1, g3), lambda i: (0, 0)),
            pl.BlockSpec((1, g3), lambda i: (0, 0)),
        ],
        out_specs=pl.BlockSpec((bn, c), lambda i: (i, 0)),
        out_shape=jax.ShapeDtypeStruct((n, c), jnp.float32),
    )(agg, h, wih_t, whh_t, bih, bhh)


def _sc_gather_sum(m, idx3, deg, nc, ns):
    npad, c = m.shape
    nw = nc * ns
    rows_per_step = idx3.shape[2]
    sub = rows_per_step // deg
    steps = idx3.shape[1]
    dst_per_w = steps * sub
    cvecs = c // _LANES
    mesh = plsc.VectorSubcoreMesh(core_axis_name="c", subcore_axis_name="s")

    @functools.partial(
        pl.kernel,
        out_type=jax.ShapeDtypeStruct((npad, c), jnp.float32),
        mesh=mesh,
        scratch_types=[
            pltpu.VMEM((steps, rows_per_step), jnp.int32),
            pltpu.VMEM((2, rows_per_step, c), jnp.float32),
            pltpu.VMEM((dst_per_w, c), jnp.float32),
            pltpu.SemaphoreType.DMA,
            pltpu.SemaphoreType.DMA,
        ],
    )
    def k(m_hbm, idx_hbm, out_hbm, idx_v, rows_v, out_v, sem0, sem1):
        wid = lax.axis_index("s") * nc + lax.axis_index("c")
        pltpu.sync_copy(idx_hbm.at[wid], idx_v)
        sems = (sem0, sem1)

        def start(g, b):
            pltpu.async_copy(m_hbm.at[idx_v.at[g]], rows_v.at[b], sems[b])

        def wait(g, b):
            pltpu.make_async_copy(m_hbm.at[idx_v.at[g]], rows_v.at[b],
                                  sems[b]).wait()

        start(0, 0)
        start(1, 1)

        def body(i, carry):
            for b in range(2):
                g = i * 2 + b
                wait(g, b)
                for d in range(sub):
                    def nbody(j, acc):
                        r = d * deg + j
                        return tuple(
                            acc[v] + rows_v[b, r, pl.ds(v * _LANES, _LANES)]
                            for v in range(cvecs))
                    acc = lax.fori_loop(
                        0, deg, nbody,
                        tuple(jnp.zeros((_LANES,), jnp.float32)
                              for _ in range(cvecs)))
                    row_out = g * sub + d
                    for v in range(cvecs):
                        out_v[row_out, pl.ds(v * _LANES, _LANES)] = acc[v]

                @pl.when(g + 2 < steps)
                def _():
                    start(g + 2, b)
            return carry

        lax.fori_loop(0, steps // 2, body, 0)
        pltpu.sync_copy(out_v, out_hbm.at[pl.ds(wid * dst_per_w, dst_per_w)])

    return k(m, idx3)


def kernel(x, edge_index, weight, W_ih, W_hh, b_ih, b_hh):
    n, c = x.shape
    deg = edge_index.shape[1]
    num_layers = weight.shape[0]
    info = plsc.get_sparse_core_info()
    nc, ns = info.num_cores, info.num_subcores
    nw = nc * ns

    rows_per_step = 128
    sub = rows_per_step // deg
    per_w = sub * nw
    steps = -(-n // per_w)
    steps += steps % 2
    npad = steps * per_w

    xp = jnp.concatenate(
        [x, jnp.zeros((npad - n, c), jnp.float32)], axis=0)
    ei = jnp.concatenate(
        [edge_index, jnp.zeros((npad - n, deg), jnp.int32)], axis=0)
    idx3 = ei.reshape(nw, steps, rows_per_step)

    wih_t = W_ih.T
    whh_t = W_hh.T
    bih = b_ih.reshape(1, -1)
    bhh = b_hh.reshape(1, -1)

    bn = 256
    for i in range(num_layers):
        m = _tc_matmul(xp, weight[i], bn)
        agg = _sc_gather_sum(m, idx3, deg, nc, ns)
        xp = _tc_gru(agg, xp, wih_t, whh_t, bih, bhh, bn)
    return xp[:n]
